# baseline (speedup 1.0000x reference)
import jax
import jax.numpy as jnp
from jax import lax
from jax.experimental import pallas as pl
from jax.experimental.pallas import tpu as pltpu
from jax.experimental.pallas import tpu_sc as plsc

F = 32
B_USER = 16384
B_ITEM = 4096
NC = 2
NS = 16
L = 16
NW = NC * NS
UPW = B_USER // NW
IPS = B_ITEM // NS
CH = 128
N_UCH = UPW // CH
N_ICH = IPS // CH
RPB = 128 // F


def _mf_body(uf, itf, uidx, iidx, out,
             uidx_v, ugidx_v, urows_v, iidx_v, igidx_v, irows_v,
             ps_v, part_v, out_v, shared, usem, isem):
  cid = lax.axis_index("c")
  sid = lax.axis_index("s")
  wid = sid * NC + cid
  ubase = wid * UPW
  zero = jnp.zeros((L,), jnp.float32)

  ucopies = []
  for t in range(N_UCH):
    pltpu.sync_copy(uidx.at[pl.ds(ubase + t * CH, CH)], uidx_v.at[t])
    for k in range(CH // L):
      raw = uidx_v[t, pl.ds(k * L, L)]
      ugidx_v[t, pl.ds(k * L, L)] = lax.shift_right_logical(raw, 2)
    ucopies.append(
        pltpu.async_copy(uf.at[ugidx_v.at[t]],
                         urows_v.at[pl.ds(t * CH, CH)], usem))

  ibase = sid * IPS
  icopies = []
  for t in range(N_ICH):
    pltpu.sync_copy(iidx.at[pl.ds(ibase + t * CH, CH)], iidx_v.at[t])
    for k in range(CH // L):
      raw = iidx_v[t, pl.ds(k * L, L)]
      igidx_v[t, pl.ds(k * L, L)] = lax.shift_right_logical(raw, 2)
    icopies.append(
        pltpu.async_copy(itf.at[igidx_v.at[t]],
                         irows_v.at[pl.ds(t * CH, CH)], isem))
  for c in icopies:
    c.wait()

  @pl.loop(0, IPS // L, init_carry=(zero, zero))
  def _item_acc(b, carry):
    a0, a1 = carry
    jv = iidx_v[b // (CH // L), pl.ds((b % (CH // L)) * L, L)]
    offs = (jv & (RPB - 1)) * F
    for l in range(L):
      row = b * L + l
      off = offs[l]
      a0 = a0 + irows_v[row, pl.ds(off, L)]
      a1 = a1 + irows_v[row, pl.ds(off + L, L)]
    return (a0, a1)
  a0, a1 = _item_acc
  part_v[pl.ds(0, L)] = a0
  part_v[pl.ds(L, L)] = a1

  pltpu.sync_copy(part_v, shared.at[pl.ds(sid * F, F)])
  plsc.subcore_barrier()
  pltpu.sync_copy(shared, ps_v)

  @pl.loop(0, NS, init_carry=(zero, zero), unroll=True)
  def _part_acc(i, carry):
    a0, a1 = carry
    return (a0 + ps_v[pl.ds(i * F, L)], a1 + ps_v[pl.ds(i * F + L, L)])
  s0, s1 = _part_acc

  for c in ucopies:
    c.wait()

  s_sc = [s0[f] for f in range(L)] + [s1[f] for f in range(L)]
  lane = lax.iota(jnp.int32, L)

  @pl.loop(0, UPW // L)
  def _dot_block(b):
    rows = b * L + lane
    rv = uidx_v[b // (CH // L), pl.ds((b % (CH // L)) * L, L)]
    cols = (rv & (RPB - 1)) * F
    acc = zero
    for f in range(F):
      col = plsc.load_gather(urows_v, [rows, cols + f])
      acc = acc + col * s_sc[f]
    out_v[pl.ds(b * L, L)] = acc

  pltpu.sync_copy(out_v, out.at[pl.ds(ubase, UPW)])


_mf_kernel = pl.kernel(
    _mf_body,
    out_type=jax.ShapeDtypeStruct((B_USER,), jnp.float32),
    mesh=plsc.VectorSubcoreMesh(core_axis_name="c", subcore_axis_name="s"),
    compiler_params=pltpu.CompilerParams(
        needs_layout_passes=False, use_tc_tiling_on_sc=True),
    scratch_types=[
        pltpu.VMEM((N_UCH, CH), jnp.int32),
        pltpu.VMEM((N_UCH, CH), jnp.int32),
        pltpu.VMEM((UPW, 128), jnp.float32),
        pltpu.VMEM((N_ICH, CH), jnp.int32),
        pltpu.VMEM((N_ICH, CH), jnp.int32),
        pltpu.VMEM((IPS, 128), jnp.float32),
        pltpu.VMEM((NS * F,), jnp.float32),
        pltpu.VMEM((F,), jnp.float32),
        pltpu.VMEM((UPW,), jnp.float32),
        pltpu.VMEM_SHARED((NS * F,), jnp.float32),
        pltpu.SemaphoreType.DMA,
        pltpu.SemaphoreType.DMA,
    ],
)


def kernel(user_factors, item_factors, user_indices, item_indices):
  uf = user_factors.reshape(-1, 128)
  itf = item_factors.reshape(-1, 128)
  return _mf_kernel(uf, itf,
                    user_indices.astype(jnp.int32),
                    item_indices.astype(jnp.int32))


# speedup vs baseline: 8.4464x; 8.4464x over previous
import jax
import jax.numpy as jnp
from jax import lax
from jax.experimental import pallas as pl
from jax.experimental.pallas import tpu as pltpu
from jax.experimental.pallas import tpu_sc as plsc

F = 32
B_USER = 16384
B_ITEM = 4096
NV = 1000000
NC = 2
NS = 16
L = 16
NW = NC * NS
IPW = B_ITEM // NW
UPW = B_USER // NW
CH = 128
N_UCH = UPW // CH
NBUF = 8

_SC_PARAMS = pltpu.CompilerParams(
    needs_layout_passes=False, use_tc_tiling_on_sc=True)
_SC_MESH = plsc.VectorSubcoreMesh(core_axis_name="c", subcore_axis_name="s")


MAXC = (NV - CH) // CH * CH
TAIL = NV // CH * CH


def _item_body(itT, iidx, s2_out, idx_sm, part_v, ps_v, shared, sem, tsem,
               tail_v, *blks):
  cid = lax.axis_index("c")
  sid = lax.axis_index("s")
  wid = sid * NC + cid
  zero = jnp.zeros((L,), jnp.float32)
  lane = lax.iota(jnp.int32, L)

  pltpu.sync_copy(iidx.at[pl.ds(wid * IPW, IPW)], idx_sm)
  pltpu.async_copy(itT.at[:, pl.ds(TAIL, NV - TAIL)], tail_v, tsem).wait()

  js = []
  for b in range(IPW // L):
    jv = idx_sm[pl.ds(b * L, L)]
    js.extend(jv[l] for l in range(L))

  def aligned_col(j):
    return pl.multiple_of(
        jnp.minimum(j & ~jnp.int32(CH - 1), jnp.int32(MAXC)), CH)

  def fire(k):
    col = aligned_col(js[k])
    return pltpu.async_copy(itT.at[:, pl.ds(col, CH)], blks[k % NBUF], sem)

  copies = [fire(k) for k in range(NBUF)]
  a0, a1 = zero, zero
  for k in range(IPW):
    copies[k % NBUF].wait()
    j = js[k]
    col = aligned_col(j)
    is_tail = j >= TAIL
    sub = jnp.full((L,), jnp.minimum(j - col, CH - 1), jnp.int32)
    tsub = jnp.full((L,), jnp.clip(j - TAIL, 0, NV - TAIL - 1), jnp.int32)
    m0 = plsc.load_gather(blks[k % NBUF], [lane, sub])
    m1 = plsc.load_gather(blks[k % NBUF], [lane + L, sub])
    t0 = plsc.load_gather(tail_v, [lane, tsub])
    t1 = plsc.load_gather(tail_v, [lane + L, tsub])
    a0 = a0 + jnp.where(is_tail, t0, m0)
    a1 = a1 + jnp.where(is_tail, t1, m1)
    if k + NBUF < IPW:
      copies[k % NBUF] = fire(k + NBUF)

  part_v[pl.ds(0, L)] = a0
  part_v[pl.ds(L, L)] = a1
  pltpu.sync_copy(part_v, shared.at[pl.ds(sid * F, F)])
  plsc.subcore_barrier()
  pltpu.sync_copy(shared, ps_v)

  @pl.loop(0, NS, init_carry=(zero, zero), unroll=True)
  def _part_acc(i, carry):
    b0, b1 = carry
    return (b0 + ps_v[pl.ds(i * F, L)], b1 + ps_v[pl.ds(i * F + L, L)])
  s0, s1 = _part_acc

  @pl.when(sid == 0)
  def _():
    part_v[pl.ds(0, L)] = s0
    part_v[pl.ds(L, L)] = s1
    pltpu.sync_copy(part_v, s2_out.at[pl.ds(cid * F, F)])


_item_kernel = pl.kernel(
    _item_body,
    out_type=jax.ShapeDtypeStruct((NC * F,), jnp.float32),
    mesh=_SC_MESH,
    compiler_params=_SC_PARAMS,
    scratch_types=[
        pltpu.VMEM((IPW,), jnp.int32),
        pltpu.VMEM((F,), jnp.float32),
        pltpu.VMEM((NS * F,), jnp.float32),
        pltpu.VMEM_SHARED((NS * F,), jnp.float32),
        pltpu.SemaphoreType.DMA,
        pltpu.SemaphoreType.DMA,
        pltpu.VMEM((F, NV - TAIL), jnp.float32),
    ] + [pltpu.VMEM((F, CH), jnp.float32) for _ in range(NBUF)],
)


BN = 32768
NB = (NV + BN - 1) // BN


def _sweep_body(s2_ref, ut_ref, y_ref):
  s = s2_ref[pl.ds(0, F)] + s2_ref[pl.ds(F, F)]
  y_ref[...] = jnp.sum(ut_ref[...] * s.reshape(F, 1), axis=0)


_sweep_kernel = pl.pallas_call(
    _sweep_body,
    out_shape=jax.ShapeDtypeStruct((NV,), jnp.float32),
    grid=(NB,),
    in_specs=[
        pl.BlockSpec((NC * F,), lambda j: (0,)),
        pl.BlockSpec((F, BN), lambda j: (0, j)),
    ],
    out_specs=pl.BlockSpec((BN,), lambda j: (j,)),
)


def _pick_body(y, uidx, out, idx_v, yv, sem):
  cid = lax.axis_index("c")
  sid = lax.axis_index("s")
  wid = sid * NC + cid
  base = wid * UPW
  copies = []
  for t in range(N_UCH):
    pltpu.sync_copy(uidx.at[pl.ds(base + t * CH, CH)], idx_v.at[t])
    copies.append(
        pltpu.async_copy(y.at[idx_v.at[t]], yv.at[pl.ds(t * CH, CH)], sem))
  for c in copies:
    c.wait()
  pltpu.sync_copy(yv, out.at[pl.ds(base, UPW)])


_pick_kernel = pl.kernel(
    _pick_body,
    out_type=jax.ShapeDtypeStruct((B_USER,), jnp.float32),
    mesh=_SC_MESH,
    compiler_params=_SC_PARAMS,
    scratch_types=[
        pltpu.VMEM((N_UCH, CH), jnp.int32),
        pltpu.VMEM((UPW,), jnp.float32),
        pltpu.SemaphoreType.DMA,
    ],
)


def kernel(user_factors, item_factors, user_indices, item_indices):
  uT = user_factors.T
  itT = item_factors.T
  s2 = _item_kernel(itT, item_indices.astype(jnp.int32))
  y = _sweep_kernel(s2, uT)
  return _pick_kernel(y, user_indices.astype(jnp.int32))


# speedup vs baseline: 8.9162x; 1.0556x over previous
import jax
import jax.numpy as jnp
from jax import lax
from jax.experimental import pallas as pl
from jax.experimental.pallas import tpu as pltpu
from jax.experimental.pallas import tpu_sc as plsc

F = 32
B_USER = 16384
B_ITEM = 4096
NV = 1000000
NC = 2
NS = 16
L = 16
NW = NC * NS
IPW = B_ITEM // NW
UPW = B_USER // NW
CH = 128
N_UCH = UPW // CH
NBUF = 16

_SC_PARAMS = pltpu.CompilerParams(
    needs_layout_passes=False, use_tc_tiling_on_sc=True)
_SC_MESH = plsc.VectorSubcoreMesh(core_axis_name="c", subcore_axis_name="s")


MAXC = (NV - CH) // CH * CH
TAIL = NV // CH * CH


def _item_body(itT, iidx, s2_out, idx_sm, part_v, ps_v, shared, sem, tsem,
               tail_v, *blks):
  cid = lax.axis_index("c")
  sid = lax.axis_index("s")
  wid = sid * NC + cid
  zero = jnp.zeros((L,), jnp.float32)
  lane = lax.iota(jnp.int32, L)

  pltpu.sync_copy(iidx.at[pl.ds(wid * IPW, IPW)], idx_sm)
  pltpu.async_copy(itT.at[:, pl.ds(TAIL, NV - TAIL)], tail_v, tsem).wait()

  js = []
  for b in range(IPW // L):
    jv = idx_sm[pl.ds(b * L, L)]
    js.extend(jv[l] for l in range(L))

  def aligned_col(j):
    return pl.multiple_of(
        jnp.minimum(j & ~jnp.int32(CH - 1), jnp.int32(MAXC)), CH)

  def fire(k):
    col = aligned_col(js[k])
    return pltpu.async_copy(itT.at[:, pl.ds(col, CH)], blks[k % NBUF], sem)

  copies = [fire(k) for k in range(NBUF)]
  a0, a1 = zero, zero
  for k in range(IPW):
    copies[k % NBUF].wait()
    j = js[k]
    col = aligned_col(j)
    is_tail = j >= TAIL
    sub = jnp.full((L,), jnp.minimum(j - col, CH - 1), jnp.int32)
    tsub = jnp.full((L,), jnp.clip(j - TAIL, 0, NV - TAIL - 1), jnp.int32)
    m0 = plsc.load_gather(blks[k % NBUF], [lane, sub])
    m1 = plsc.load_gather(blks[k % NBUF], [lane + L, sub])
    t0 = plsc.load_gather(tail_v, [lane, tsub])
    t1 = plsc.load_gather(tail_v, [lane + L, tsub])
    a0 = a0 + jnp.where(is_tail, t0, m0)
    a1 = a1 + jnp.where(is_tail, t1, m1)
    if k + NBUF < IPW:
      copies[k % NBUF] = fire(k + NBUF)

  part_v[pl.ds(0, L)] = a0
  part_v[pl.ds(L, L)] = a1
  pltpu.sync_copy(part_v, shared.at[pl.ds(sid * F, F)])
  plsc.subcore_barrier()
  pltpu.sync_copy(shared, ps_v)

  @pl.loop(0, NS, init_carry=(zero, zero), unroll=True)
  def _part_acc(i, carry):
    b0, b1 = carry
    return (b0 + ps_v[pl.ds(i * F, L)], b1 + ps_v[pl.ds(i * F + L, L)])
  s0, s1 = _part_acc

  @pl.when(sid == 0)
  def _():
    part_v[pl.ds(0, L)] = s0
    part_v[pl.ds(L, L)] = s1
    pltpu.sync_copy(part_v, s2_out.at[pl.ds(cid * F, F)])


_item_kernel = pl.kernel(
    _item_body,
    out_type=jax.ShapeDtypeStruct((NC * F,), jnp.float32),
    mesh=_SC_MESH,
    compiler_params=_SC_PARAMS,
    scratch_types=[
        pltpu.VMEM((IPW,), jnp.int32),
        pltpu.VMEM((F,), jnp.float32),
        pltpu.VMEM((NS * F,), jnp.float32),
        pltpu.VMEM_SHARED((NS * F,), jnp.float32),
        pltpu.SemaphoreType.DMA,
        pltpu.SemaphoreType.DMA,
        pltpu.VMEM((F, NV - TAIL), jnp.float32),
    ] + [pltpu.VMEM((F, CH), jnp.float32) for _ in range(NBUF)],
)


BN = 32768
NB = (NV + BN - 1) // BN


def _sweep_body(s2_ref, ut_ref, y_ref):
  s = s2_ref[pl.ds(0, F)] + s2_ref[pl.ds(F, F)]
  y_ref[...] = jnp.dot(s.reshape(1, F), ut_ref[...],
                       preferred_element_type=jnp.float32).reshape(BN)


_sweep_kernel = pl.pallas_call(
    _sweep_body,
    out_shape=jax.ShapeDtypeStruct((NV,), jnp.float32),
    grid=(NB,),
    in_specs=[
        pl.BlockSpec((NC * F,), lambda j: (0,)),
        pl.BlockSpec((F, BN), lambda j: (0, j)),
    ],
    out_specs=pl.BlockSpec((BN,), lambda j: (j,)),
)


def _pick_body(y, uidx, out, idx_v, yv, sem):
  cid = lax.axis_index("c")
  sid = lax.axis_index("s")
  wid = sid * NC + cid
  base = wid * UPW
  copies = []
  for t in range(N_UCH):
    pltpu.sync_copy(uidx.at[pl.ds(base + t * CH, CH)], idx_v.at[t])
    copies.append(
        pltpu.async_copy(y.at[idx_v.at[t]], yv.at[pl.ds(t * CH, CH)], sem))
  for c in copies:
    c.wait()
  pltpu.sync_copy(yv, out.at[pl.ds(base, UPW)])


_pick_kernel = pl.kernel(
    _pick_body,
    out_type=jax.ShapeDtypeStruct((B_USER,), jnp.float32),
    mesh=_SC_MESH,
    compiler_params=_SC_PARAMS,
    scratch_types=[
        pltpu.VMEM((N_UCH, CH), jnp.int32),
        pltpu.VMEM((UPW,), jnp.float32),
        pltpu.SemaphoreType.DMA,
    ],
)


def kernel(user_factors, item_factors, user_indices, item_indices):
  uT = user_factors.T
  itT = item_factors.T
  s2 = _item_kernel(itT, item_indices.astype(jnp.int32))
  y = _sweep_kernel(s2, uT)
  return _pick_kernel(y, user_indices.astype(jnp.int32))


# speedup vs baseline: 9.1862x; 1.0303x over previous
import jax
import jax.numpy as jnp
from jax import lax
from jax.experimental import pallas as pl
from jax.experimental.pallas import tpu as pltpu
from jax.experimental.pallas import tpu_sc as plsc

F = 32
B_USER = 16384
B_ITEM = 4096
NV = 1000000
NC = 2
NS = 16
L = 16
NW = NC * NS
IPW = B_ITEM // NW
UPW = B_USER // NW
CH = 128
N_UCH = UPW // CH
NBUF = 16

_SC_PARAMS = pltpu.CompilerParams(
    needs_layout_passes=False, use_tc_tiling_on_sc=True)
_SC_MESH = plsc.VectorSubcoreMesh(core_axis_name="c", subcore_axis_name="s")


MAXC = (NV - CH) // CH * CH
TAIL = NV // CH * CH


def _item_body(itT, iidx, s2_out, idx_sm, part_v, ps_v, shared, sem, tsem,
               tail_v, *blks):
  cid = lax.axis_index("c")
  sid = lax.axis_index("s")
  wid = sid * NC + cid
  zero = jnp.zeros((L,), jnp.float32)
  lane = lax.iota(jnp.int32, L)

  pltpu.sync_copy(iidx.at[pl.ds(wid * IPW, IPW)], idx_sm)
  pltpu.async_copy(itT.at[:, pl.ds(TAIL, NV - TAIL)], tail_v, tsem).wait()

  js = []
  for b in range(IPW // L):
    jv = idx_sm[pl.ds(b * L, L)]
    js.extend(jv[l] for l in range(L))

  def aligned_col(j):
    return pl.multiple_of(
        jnp.minimum(j & ~jnp.int32(CH - 1), jnp.int32(MAXC)), CH)

  def fire(k):
    col = aligned_col(js[k])
    return pltpu.async_copy(itT.at[:, pl.ds(col, CH)], blks[k % NBUF], sem)

  copies = [fire(k) for k in range(NBUF)]
  a0, a1 = zero, zero
  for k in range(IPW):
    copies[k % NBUF].wait()
    j = js[k]
    col = aligned_col(j)
    is_tail = j >= TAIL
    sub = jnp.full((L,), jnp.minimum(j - col, CH - 1), jnp.int32)
    tsub = jnp.full((L,), jnp.clip(j - TAIL, 0, NV - TAIL - 1), jnp.int32)
    m0 = plsc.load_gather(blks[k % NBUF], [lane, sub])
    m1 = plsc.load_gather(blks[k % NBUF], [lane + L, sub])
    t0 = plsc.load_gather(tail_v, [lane, tsub])
    t1 = plsc.load_gather(tail_v, [lane + L, tsub])
    a0 = a0 + jnp.where(is_tail, t0, m0)
    a1 = a1 + jnp.where(is_tail, t1, m1)
    if k + NBUF < IPW:
      copies[k % NBUF] = fire(k + NBUF)

  part_v[pl.ds(0, L)] = a0
  part_v[pl.ds(L, L)] = a1
  pltpu.sync_copy(part_v, shared.at[pl.ds(sid * F, F)])
  plsc.subcore_barrier()
  pltpu.sync_copy(shared, ps_v)

  @pl.loop(0, NS, init_carry=(zero, zero), unroll=True)
  def _part_acc(i, carry):
    b0, b1 = carry
    return (b0 + ps_v[pl.ds(i * F, L)], b1 + ps_v[pl.ds(i * F + L, L)])
  s0, s1 = _part_acc

  @pl.when(sid == 0)
  def _():
    part_v[pl.ds(0, L)] = s0
    part_v[pl.ds(L, L)] = s1
    pltpu.sync_copy(part_v, s2_out.at[pl.ds(cid * F, F)])


_item_kernel = pl.kernel(
    _item_body,
    out_type=jax.ShapeDtypeStruct((NC * F,), jnp.float32),
    mesh=_SC_MESH,
    compiler_params=_SC_PARAMS,
    scratch_types=[
        pltpu.VMEM((IPW,), jnp.int32),
        pltpu.VMEM((F,), jnp.float32),
        pltpu.VMEM((NS * F,), jnp.float32),
        pltpu.VMEM_SHARED((NS * F,), jnp.float32),
        pltpu.SemaphoreType.DMA,
        pltpu.SemaphoreType.DMA,
        pltpu.VMEM((F, NV - TAIL), jnp.float32),
    ] + [pltpu.VMEM((F, CH), jnp.float32) for _ in range(NBUF)],
)


BN = 131072
NB = (NV + BN - 1) // BN


def _sweep_body(s2_ref, ut_ref, y_ref):
  s = s2_ref[pl.ds(0, F)] + s2_ref[pl.ds(F, F)]
  y_ref[...] = jnp.dot(s.reshape(1, F), ut_ref[...],
                       preferred_element_type=jnp.float32).reshape(BN)


_sweep_kernel = pl.pallas_call(
    _sweep_body,
    out_shape=jax.ShapeDtypeStruct((NV,), jnp.float32),
    grid=(NB,),
    in_specs=[
        pl.BlockSpec((NC * F,), lambda j: (0,)),
        pl.BlockSpec((F, BN), lambda j: (0, j)),
    ],
    out_specs=pl.BlockSpec((BN,), lambda j: (j,)),
)


def _pick_body(y, uidx, out, idx_v, yv, sem):
  cid = lax.axis_index("c")
  sid = lax.axis_index("s")
  wid = sid * NC + cid
  base = wid * UPW
  copies = []
  for t in range(N_UCH):
    pltpu.sync_copy(uidx.at[pl.ds(base + t * CH, CH)], idx_v.at[t])
    copies.append(
        pltpu.async_copy(y.at[idx_v.at[t]], yv.at[pl.ds(t * CH, CH)], sem))
  for c in copies:
    c.wait()
  pltpu.sync_copy(yv, out.at[pl.ds(base, UPW)])


_pick_kernel = pl.kernel(
    _pick_body,
    out_type=jax.ShapeDtypeStruct((B_USER,), jnp.float32),
    mesh=_SC_MESH,
    compiler_params=_SC_PARAMS,
    scratch_types=[
        pltpu.VMEM((N_UCH, CH), jnp.int32),
        pltpu.VMEM((UPW,), jnp.float32),
        pltpu.SemaphoreType.DMA,
    ],
)


def kernel(user_factors, item_factors, user_indices, item_indices):
  uT = user_factors.T
  itT = item_factors.T
  s2 = _item_kernel(itT, item_indices.astype(jnp.int32))
  y = _sweep_kernel(s2, uT)
  return _pick_kernel(y, user_indices.astype(jnp.int32))
